# Initial kernel scaffold; baseline (speedup 1.0000x reference)
import jax
import jax.numpy as jnp
from jax.experimental import pallas as pl


def kernel(table, query, q_table_dgl_graph, q_table_table_embs, dgl_graph, table_embs, q_feat, params):
    raise NotImplementedError("write your pallas kernel here")


# speedup vs baseline: 3.7216x; 3.7216x over previous
import functools

import jax
import jax.numpy as jnp
from jax import lax
from jax.experimental import pallas as pl
from jax.experimental.pallas import tpu as pltpu
from jax.experimental.pallas import tpu_sc as plsc

N = 10000
E = 160000
D = 300
LAYERS = 4

NS = 16
DH = 160
EB = 64
CHB = 8
CE = CHB * EB
NC = 20
EPT = NC * CE
E_PAD = EPT * NS
N_PAD = 10240
RPT = N_PAD // NS
MR_SRC = 0
MR_PH = 8
MR_DST = 9
META_R = 17
ROWS_PER_BLK = 1000


def _leaky(v):
    return jnp.where(v > 0, v, 0.2 * v)


def _lane_splat(vec16, lane):
    return lax.gather(
        vec16, jnp.full((16, 1), lane, jnp.int32),
        dimension_numbers=lax.GatherDimensionNumbers(
            offset_dims=(), collapsed_slice_dims=(0,), start_index_map=(0,)),
        slice_sizes=(1,), mode=lax.GatherScatterMode.PROMISE_IN_BOUNDS)


_sc_mesh = plsc.VectorSubcoreMesh(core_axis_name="c", subcore_axis_name="s",
                                  num_cores=2, num_subcores=NS)


@functools.partial(
    pl.kernel,
    mesh=_sc_mesh,
    compiler_params=pltpu.CompilerParams(use_tc_tiling_on_sc=False,
                                         needs_layout_passes=False),
    out_type=jax.ShapeDtypeStruct((2, N_PAD, DH), jnp.float32),
    scratch_types=[
        pltpu.VMEM((META_R, EB), jnp.int32),
        pltpu.VMEM((META_R, EB), jnp.int32),
        pltpu.VMEM((EB, DH), jnp.float32),
        pltpu.VMEM((EB, DH), jnp.float32),
        pltpu.VMEM((CHB, EB), jnp.float32),
        pltpu.VMEM((CHB, EB), jnp.float32),
        pltpu.VMEM((CHB, EB), jnp.float32),
        pltpu.VMEM((16,), jnp.float32),
        pltpu.VMEM_SHARED((N_PAD, DH), jnp.float32),
        pltpu.SemaphoreType.DMA,
        pltpu.SemaphoreType.DMA,
        pltpu.SemaphoreType.DMA,
    ],
)
def _sc_edge_agg(hw2_hbm, meta_hbm, as_hbm, ad_hbm, g_hbm, zeros_hbm, out_hbm,
                 meta_a, meta_b, rows_a, rows_b, sbuf, dbuf, wbuf, gv,
                 acc, gsem, msem, wsem):
    c = lax.axis_index("c")
    s = lax.axis_index("s")

    pltpu.sync_copy(zeros_hbm, acc.at[pl.ds(s * RPT, RPT)])
    pltpu.sync_copy(meta_hbm.at[s, 0], meta_a)
    pltpu.sync_copy(g_hbm, gv)
    plsc.subcore_barrier()

    def compute_w(meta):
        for b in range(CHB):
            pltpu.async_copy(as_hbm.at[meta.at[MR_SRC + b]], sbuf.at[b], wsem)
            pltpu.async_copy(ad_hbm.at[meta.at[MR_DST + b]], dbuf.at[b], wsem)
        for b in range(CHB):
            pltpu.make_async_copy(as_hbm.at[meta.at[0]], sbuf.at[b], wsem).wait()
            pltpu.make_async_copy(ad_hbm.at[meta.at[0]], dbuf.at[b], wsem).wait()
        gvv = gv[...]
        for b in range(CHB):
            for grp in range(EB // 16):
                sl = pl.ds(grp * 16, 16)
                x = sbuf[b, sl] + dbuf[b, sl]
                x = jnp.where(x > 0, x, 0.2 * x)
                wbuf[b, sl] = jnp.exp(x - gvv)

    def scale(meta, b, rows):
        def grp_body(grp, carry):
            w16 = wbuf[b, pl.ds(grp * 16, 16)]
            for jj in range(16):
                wsplat = _lane_splat(w16, jj)
                j = grp * 16 + jj
                for kk in range(DH // 16):
                    sl = pl.ds(kk * 16, 16)
                    rows[j, sl] = rows[j, sl] * wsplat
            return carry

        lax.fori_loop(0, EB // 16, grp_body, 0)

    hw2_c = hw2_hbm.at[c]

    def gather(meta, b, rows):
        pltpu.async_copy(hw2_c.at[meta.at[b]], rows, gsem)

    def gather_wait(meta, rows):
        pltpu.make_async_copy(hw2_c.at[meta.at[0]], rows, gsem).wait()

    def meta_wait(meta):
        pltpu.make_async_copy(meta_hbm.at[s, 0], meta, msem).wait()

    def process_chunk(meta, b, carry):
        b0 = 2 * b
        gather_wait(meta, rows_a)
        gather(meta, b0 + 1, rows_b)
        scale(meta, b0, rows_a)
        pltpu.sync_copy(rows_a, acc.at[meta.at[MR_DST + b0]], add=True)
        gather_wait(meta, rows_b)
        gather(meta, b0 + 2, rows_a)
        scale(meta, b0 + 1, rows_b)
        pltpu.sync_copy(rows_b, acc.at[meta.at[MR_DST + b0 + 1]], add=True)
        return carry

    def chunk_pair(cc, carry):
        ch0 = 2 * cc
        pltpu.async_copy(meta_hbm.at[s, ch0 + 1], meta_b, msem)
        gather(meta_a, 0, rows_a)
        compute_w(meta_a)
        lax.fori_loop(0, CHB // 2,
                      functools.partial(process_chunk, meta_a), 0)
        gather_wait(meta_a, rows_a)
        meta_wait(meta_b)
        pltpu.async_copy(meta_hbm.at[s, ch0 + 2], meta_a, msem)
        gather(meta_b, 0, rows_a)
        compute_w(meta_b)
        lax.fori_loop(0, CHB // 2,
                      functools.partial(process_chunk, meta_b), 0)
        gather_wait(meta_b, rows_a)
        meta_wait(meta_a)
        return carry

    lax.fori_loop(0, NC // 2, chunk_pair, 0)

    plsc.subcore_barrier()
    pltpu.sync_copy(acc.at[pl.ds(s * RPT, RPT)],
                    out_hbm.at[c, pl.ds(s * RPT, RPT)])


DR = D - DH


def _unnorm(in2):
    agg = jnp.concatenate([in2[0], in2[1][:, :DR]], axis=1)
    den = in2[1][:, DR]
    return agg / (den[:, None] + 1e-9)


def _layer_body(in_ref, w_ref, aa_ref, out2_ref, aout_ref, *, apply_act):
    h = _unnorm(in_ref[...])
    if apply_act:
        h = _leaky(h)
    hw = lax.dot_general(h, w_ref[...], (((1,), (0,)), ((), ())),
                         preferred_element_type=jnp.float32)
    aout_ref[...] = lax.dot_general(
        hw, aa_ref[...], (((1,), (0,)), ((), ())),
        preferred_element_type=jnp.float32)
    rows = hw.shape[0]
    out2_ref[0] = hw[:, :DH]
    out2_ref[1] = jnp.concatenate(
        [hw[:, DH:], jnp.ones((rows, 1), jnp.float32),
         jnp.zeros((rows, DH - DR - 1), jnp.float32)], axis=1)


def _layer_stage(in2, W, a_src, a_dst, apply_act):
    nblk = N // ROWS_PER_BLK
    aa = jnp.stack([a_src, a_dst], axis=1)
    hw2, asd = pl.pallas_call(
        functools.partial(_layer_body, apply_act=apply_act),
        grid=(nblk,),
        in_specs=[
            pl.BlockSpec((2, ROWS_PER_BLK, DH), lambda i: (0, i, 0)),
            pl.BlockSpec((D, D), lambda i: (0, 0)),
            pl.BlockSpec((D, 2), lambda i: (0, 0)),
        ],
        out_specs=[
            pl.BlockSpec((2, ROWS_PER_BLK, DH), lambda i: (0, i, 0)),
            pl.BlockSpec((ROWS_PER_BLK, 2), lambda i: (i, 0)),
        ],
        out_shape=[
            jax.ShapeDtypeStruct((2, N, DH), jnp.float32),
            jax.ShapeDtypeStruct((N, 2), jnp.float32),
        ],
    )(in2, W, aa)
    return hw2, asd[:, 0], asd[:, 1]


def _head_body(ina_ref, inb_ref, pw_ref, pb_ref, lng_ref, lnb_ref,
               w_ref, b_ref, out_ref):
    i = pl.program_id(0)

    def proj_ln(in2):
        cfeat = _unnorm(in2[...])
        t = lax.dot_general(cfeat, pw_ref[...], (((1,), (0,)), ((), ())),
                            preferred_element_type=jnp.float32) + pb_ref[...]
        mu = t.mean(-1, keepdims=True)
        var = ((t - mu) ** 2).mean(-1, keepdims=True)
        return (t - mu) / jnp.sqrt(var + 1e-5) * lng_ref[...] + lnb_ref[...]

    ta = proj_ln(ina_ref)
    tb = proj_ln(inb_ref)
    x = jnp.concatenate([ta, tb, ta - tb, ta * tb], axis=1)
    h = jnp.tanh(
        lax.dot_general(x, w_ref[...], (((1,), (0,)), ((), ())),
                        preferred_element_type=jnp.float32)
        + b_ref[...]
    )
    m = jnp.max(h, axis=0, keepdims=True)

    @pl.when(i == 0)
    def _():
        out_ref[...] = m

    @pl.when(i > 0)
    def _():
        out_ref[...] = jnp.maximum(out_ref[...], m)


def _match_head(in2a, in2b, p):
    nblk = N // ROWS_PER_BLK
    vec = pl.BlockSpec((1, 4 * D), lambda i: (0, 0))
    rep = pl.pallas_call(
        _head_body,
        grid=(nblk,),
        in_specs=[
            pl.BlockSpec((2, ROWS_PER_BLK, DH), lambda i: (0, i, 0)),
            pl.BlockSpec((2, ROWS_PER_BLK, DH), lambda i: (0, i, 0)),
            pl.BlockSpec((D, D), lambda i: (0, 0)),
            pl.BlockSpec((1, D), lambda i: (0, 0)),
            pl.BlockSpec((1, D), lambda i: (0, 0)),
            pl.BlockSpec((1, D), lambda i: (0, 0)),
            pl.BlockSpec((4 * D, 4 * D), lambda i: (0, 0)),
            vec,
        ],
        out_specs=vec,
        out_shape=jax.ShapeDtypeStruct((1, 4 * D), jnp.float32),
    )(in2a, in2b, p["proj_W"], p["proj_b"].reshape(1, -1),
      p["ln_g"].reshape(1, -1), p["ln_b"].reshape(1, -1),
      p["dr_W"], p["dr_b"].reshape(1, -1))
    return rep


def _mlp_body(rep_ref, w1_ref, b1_ref, w2_ref, b2_ref, out_ref):
    h1 = _leaky(
        lax.dot_general(rep_ref[...], w1_ref[...], (((1,), (0,)), ((), ())),
                        preferred_element_type=jnp.float32) + b1_ref[...])
    out_ref[...] = lax.dot_general(
        h1, w2_ref[...], (((1,), (0,)), ((), ())),
        preferred_element_type=jnp.float32) + b2_ref[...]


def _final_mlp(rep, p):
    out = pl.pallas_call(
        _mlp_body,
        out_shape=jax.ShapeDtypeStruct((1, 1), jnp.float32),
    )(rep, p["r_W1"], p["r_b1"].reshape(1, -1),
      p["r_W2"], p["r_b2"].reshape(1, -1))
    return out.reshape(1)


def _pack_x(x):
    lo = x[:, :DH]
    hi = jnp.concatenate(
        [x[:, DH:], jnp.ones((N, 1), jnp.float32),
         jnp.zeros((N, DH - DR - 1), jnp.float32)], axis=1)
    return jnp.stack([lo, hi])


def _prep_edges(ei):
    src = ei[0].astype(jnp.int32)
    dst = ei[1].astype(jnp.int32)
    pad = E_PAD - E
    src_p = jnp.concatenate([src, jnp.zeros((pad,), jnp.int32)])
    dst_p = jnp.concatenate([dst, jnp.full((pad,), N, jnp.int32)])
    return src_p, dst_p


def _pack_meta(src_p, dst_p):
    src4 = src_p.reshape(NS, NC, CHB, EB)
    dst4 = dst_p.reshape(NS, NC, CHB, EB)
    ph = jnp.zeros((NS, NC, 1, EB), jnp.int32)
    meta = jnp.concatenate([src4, ph, dst4], axis=2)
    phc = jnp.zeros((NS, 1, META_R, EB), jnp.int32)
    return jnp.concatenate([meta, phc], axis=1)


def _gat(x, edges, p):
    src_p, dst_p = edges
    meta = _pack_meta(src_p, dst_p)
    zeros = jnp.zeros((RPT, DH), jnp.float32)
    in2 = _pack_x(x)
    for l in range(LAYERS):
        hw2, a_s, a_d = _layer_stage(
            in2, p["W%d" % l], p["as%d" % l], p["ad%d" % l], apply_act=l > 0)
        g = _leaky(jnp.max(a_s) + jnp.max(a_d))
        gvec = jnp.full((16,), g, jnp.float32)
        pad = jnp.zeros((N_PAD - N,), jnp.float32)
        agg2 = _sc_edge_agg(hw2, meta, jnp.concatenate([a_s, pad]),
                            jnp.concatenate([a_d, pad]), gvec, zeros)
        in2 = agg2[:, :N]
    return in2


def kernel(table, query, q_table_dgl_graph, q_table_table_embs, dgl_graph, table_embs, q_feat, params):
    p = params
    in2a = _gat(q_table_table_embs, _prep_edges(q_table_dgl_graph), p)
    in2b = _gat(table_embs, _prep_edges(dgl_graph), p)
    rep = _match_head(in2a, in2b, p)
    return _final_mlp(rep, p)
